# baseline (speedup 1.0000x reference)
import functools

import jax
import jax.numpy as jnp
from jax.experimental import pallas as pl
from jax.experimental.pallas import tpu as pltpu
from jax.experimental.pallas import tpu_sc as plsc

_NUM_STEPS = 3
_TOPK = 8
_TR = 512

_NW = 32
_CHUNK = 128
_RING = 8


def _f32dot(a, b):
    return jax.lax.dot_general(a, b, (((1,), (0,)), ((), ())),
                               preferred_element_type=jnp.float32)


def _norm_keys(st, Wnp, bnp):
    keys = _f32dot(st, Wnp) + bnp
    nrm = jnp.sqrt(jnp.sum(keys * keys, axis=1, keepdims=True))
    return keys / jnp.maximum(nrm, 1e-12)


def _topk_tile(knt, kn, n, base):
    sim = jax.lax.dot_general(knt, kn, (((1,), (1,)), ((), ())),
                              preferred_element_type=jnp.float32)
    iota = jax.lax.broadcasted_iota(jnp.int32, sim.shape, 1)
    iota_col = jax.lax.broadcasted_iota(jnp.int32, (n, 1), 0).astype(
        jnp.float32)
    self_col = jax.lax.broadcasted_iota(jnp.int32, (sim.shape[0], 1), 0) + base
    s = jnp.where(iota == self_col, -jnp.inf, sim)
    cols = [self_col]
    for _j in range(_TOPK - 1):
        m = jnp.max(s, axis=1, keepdims=True)
        hit = s == m
        oh = jnp.where(hit, 1.0, 0.0)
        pos = jax.lax.dot_general(
            oh, iota_col, (((1,), (0,)), ((), ())),
            precision=jax.lax.Precision.HIGHEST,
            preferred_element_type=jnp.float32)
        am = jnp.minimum(pos.astype(jnp.int32), n - 1)
        cols.append(am)
        if _j + 2 < _TOPK:
            s = jnp.where(hit, -jnp.inf, s)
    return jnp.concatenate(cols, axis=1)


def _mlp_update(stt, nsum_t, Wnp, bnp, W1a, W1b, b1, gamma, beta, W2, b2):
    agg = _f32dot(nsum_t * (1.0 / _TOPK), Wnp) + bnp
    h = _f32dot(stt, W1a) + _f32dot(agg, W1b) + b1
    mu = jnp.mean(h, axis=1, keepdims=True)
    var = jnp.mean((h - mu) ** 2, axis=1, keepdims=True)
    hn = (h - mu) * jax.lax.rsqrt(var + 1e-5) * gamma + beta
    a = hn * (1.0 / (1.0 + jnp.exp(-hn)))
    return stt + _f32dot(a, W2) + b2


def _idx_body(x_ref, Wnp_ref, bnp_ref, idx_ref):
    b = pl.program_id(0)
    n = x_ref.shape[1]
    st = x_ref[0]
    kn = _norm_keys(st, Wnp_ref[...], bnp_ref[...])
    for rt in range(n // _TR):
        knt = kn[rt * _TR:(rt + 1) * _TR]
        idx_ref[0, pl.ds(rt * _TR, _TR), :] = (
            _topk_tile(knt, kn, n, rt * _TR) + b * n)


def _update_idx_body(x_ref, nsum_ref, Wnp_ref, bnp_ref, W1a_ref, W1b_ref,
                     b1_ref, gamma_ref, beta_ref, W2_ref, b2_ref,
                     newstate_ref, idx_ref):
    b = pl.program_id(0)
    n = x_ref.shape[1]
    st = x_ref[0]
    ns = nsum_ref[0]
    tiles = []
    for rt in range(n // _TR):
        sl = slice(rt * _TR, (rt + 1) * _TR)
        tiles.append(_mlp_update(st[sl], ns[sl], Wnp_ref[...], bnp_ref[...],
                                 W1a_ref[...], W1b_ref[...], b1_ref[...],
                                 gamma_ref[...], beta_ref[...], W2_ref[...],
                                 b2_ref[...]))
    newst = jnp.concatenate(tiles, axis=0)
    newstate_ref[0] = newst
    kn = _norm_keys(newst, Wnp_ref[...], bnp_ref[...])
    for rt in range(n // _TR):
        knt = kn[rt * _TR:(rt + 1) * _TR]
        idx_ref[0, pl.ds(rt * _TR, _TR), :] = (
            _topk_tile(knt, kn, n, rt * _TR) + b * n)


def _update_readout_body(x_ref, nsum_ref, Wnp_ref, bnp_ref, W1a_ref, W1b_ref,
                         b1_ref, gamma_ref, beta_ref, W2_ref, b2_ref,
                         Wo_ref, bo_ref, out_ref):
    n = x_ref.shape[1]
    st = x_ref[0]
    ns = nsum_ref[0]
    acc = jnp.zeros((1, st.shape[1]), jnp.float32)
    for rt in range(n // _TR):
        sl = slice(rt * _TR, (rt + 1) * _TR)
        newt = _mlp_update(st[sl], ns[sl], Wnp_ref[...], bnp_ref[...],
                           W1a_ref[...], W1b_ref[...], b1_ref[...],
                           gamma_ref[...], beta_ref[...], W2_ref[...],
                           b2_ref[...])
        acc = acc + jnp.sum(newt, axis=0, keepdims=True)
    out_ref[0] = _f32dot(acc * (1.0 / n), Wo_ref[...]) + bo_ref[...]


def _sc_gather_body(table_hbm, gidx_hbm, out_hbm, gidx_v, rows_v, obuf_v,
                    *sems):
    nch = gidx_v.shape[0]
    k = _TOPK
    per_w = nch * _CHUNK // k
    c = jax.lax.axis_index("c")
    s = jax.lax.axis_index("s")
    wid = s * 2 + c
    base = wid * per_w
    pltpu.sync_copy(gidx_hbm.at[wid], gidx_v)
    copies = [None] * nch

    def fire(ch):
        copies[ch] = pltpu.async_copy(
            table_hbm.at[gidx_v.at[ch]], rows_v.at[ch % _RING],
            sems[ch % _RING])

    for ch in range(min(_RING, nch)):
        fire(ch)
    for ch in range(nch):
        copies[ch].wait()
        rows = rows_v.at[ch % _RING]

        def body(o, carry):
            r0 = o * k
            for col in range(obuf_v.shape[1] // 16):
                sl = pl.ds(col * 16, 16)
                v = rows[r0, sl]
                for j in range(1, k):
                    v = v + rows[r0 + j, sl]
                obuf_v[o, sl] = v
            return carry

        jax.lax.fori_loop(0, _CHUNK // k, body, 0)
        pltpu.sync_copy(obuf_v, out_hbm.at[pl.ds(base + ch * (_CHUNK // k),
                                                 _CHUNK // k)])
        if ch + _RING < nch:
            fire(ch + _RING)


def _sc_gather_sum(table, gidx):
    rows, d = table.shape
    nch = rows * _TOPK // (_NW * _CHUNK)
    mesh = plsc.VectorSubcoreMesh(core_axis_name="c", subcore_axis_name="s")
    return pl.kernel(
        _sc_gather_body,
        mesh=mesh,
        compiler_params=pltpu.CompilerParams(use_tc_tiling_on_sc=False),
        out_type=jax.ShapeDtypeStruct((rows, d), jnp.float32),
        scratch_types=[
            pltpu.VMEM((nch, _CHUNK), jnp.int32),
            pltpu.VMEM((_RING, _CHUNK, d), jnp.float32),
            pltpu.VMEM((_CHUNK // _TOPK, d), jnp.float32),
        ] + [pltpu.SemaphoreType.DMA] * _RING,
    )(table, gidx)


@jax.jit
def kernel(x, W_np, b_np, W1, b1, gamma, beta, W2, b2, Wo, bo):
    B, N, D = x.shape
    H = W_np.shape[1]
    O = Wo.shape[1]
    W1a = W1[:D]
    W1b = W1[D:]
    row = lambda v: v.reshape(1, -1)
    bnp, b1r, gr, br, b2r, bor = (row(b_np), row(b1), row(gamma), row(beta),
                                  row(b2), row(bo))

    full = lambda shape: pl.BlockSpec(shape, lambda b: (0,) * len(shape))
    bspec = lambda shape: pl.BlockSpec((1,) + shape,
                                       lambda b: (b,) + (0,) * len(shape))
    wspecs = [full((D, H)), full((1, H)), full((D, H)), full((H, H)),
              full((1, H)), full((1, H)), full((1, H)), full((H, D)),
              full((1, D))]

    idx_call = pl.pallas_call(
        _idx_body, grid=(B,),
        in_specs=[bspec((N, D)), full((D, H)), full((1, H))],
        out_specs=bspec((N, _TOPK)),
        out_shape=jax.ShapeDtypeStruct((B, N, _TOPK), jnp.int32),
    )
    upd_idx_call = pl.pallas_call(
        _update_idx_body, grid=(B,),
        in_specs=[bspec((N, D)), bspec((N, D))] + wspecs,
        out_specs=[bspec((N, D)), bspec((N, _TOPK))],
        out_shape=[jax.ShapeDtypeStruct((B, N, D), jnp.float32),
                   jax.ShapeDtypeStruct((B, N, _TOPK), jnp.int32)],
    )
    upd_out_call = pl.pallas_call(
        _update_readout_body, grid=(B,),
        in_specs=[bspec((N, D)), bspec((N, D))] + wspecs +
                 [full((D, O)), full((1, O))],
        out_specs=bspec((1, O)),
        out_shape=jax.ShapeDtypeStruct((B, 1, O), jnp.float32),
    )

    def gather(state, idx):
        gidx = idx.reshape(_NW, B * N * _TOPK // (_NW * _CHUNK), _CHUNK)
        nsum = _sc_gather_sum(state.reshape(B * N, D), gidx)
        return nsum.reshape(B, N, D)

    state = x
    idx = idx_call(x, W_np, bnp)
    for _step in range(_NUM_STEPS - 1):
        nsum = gather(state, idx)
        state, idx = upd_idx_call(state, nsum, W_np, bnp, W1a, W1b, b1r,
                                  gr, br, W2, b2r)
    nsum = gather(state, idx)
    out = upd_out_call(state, nsum, W_np, bnp, W1a, W1b, b1r, gr, br, W2,
                       b2r, Wo, bor)
    return out.reshape(B, O)


# speedup vs baseline: 1.9662x; 1.9662x over previous
import functools

import jax
import jax.numpy as jnp
from jax.experimental import pallas as pl
from jax.experimental.pallas import tpu as pltpu
from jax.experimental.pallas import tpu_sc as plsc

_NUM_STEPS = 3
_TOPK = 8
_TR = 512

_NW = 32
_CHUNK = 128
_RING = 8


def _f32dot(a, b):
    return jax.lax.dot_general(a, b, (((1,), (0,)), ((), ())),
                               preferred_element_type=jnp.float32)


def _norm_keys(st, Wnp, bnp):
    keys = _f32dot(st, Wnp) + bnp
    nrm = jnp.sqrt(jnp.sum(keys * keys, axis=1, keepdims=True))
    return keys / jnp.maximum(nrm, 1e-12)


def _topk_tile(knt, kn, n, base):
    sim = jax.lax.dot_general(knt, kn, (((1,), (1,)), ((), ())),
                              preferred_element_type=jnp.float32)
    iota = jax.lax.broadcasted_iota(jnp.int32, sim.shape, 1)
    iota_i = jax.lax.broadcasted_iota(jnp.int32, (n, 1), 0)
    hi_col = (iota_i // 64).astype(jnp.float32)
    lo_col = (iota_i % 64).astype(jnp.float32)
    self_col = jax.lax.broadcasted_iota(jnp.int32, (sim.shape[0], 1), 0) + base
    s = jnp.where(iota == self_col, -jnp.inf, sim)
    cols = [self_col]
    for _j in range(_TOPK - 1):
        m = jnp.max(s, axis=1, keepdims=True)
        hit = s == m
        oh = jnp.where(hit, 1.0, 0.0)
        pos = _f32dot(oh, hi_col) * 64.0 + _f32dot(oh, lo_col)
        am = jnp.minimum(pos.astype(jnp.int32), n - 1)
        cols.append(am)
        if _j + 2 < _TOPK:
            s = jnp.where(hit, -jnp.inf, s)
    return jnp.concatenate(cols, axis=1)


def _mlp_update(stt, nsum_t, Wnp, bnp, W1a, W1b, b1, gamma, beta, W2, b2):
    agg = _f32dot(nsum_t * (1.0 / _TOPK), Wnp) + bnp
    h = _f32dot(stt, W1a) + _f32dot(agg, W1b) + b1
    mu = jnp.mean(h, axis=1, keepdims=True)
    var = jnp.mean((h - mu) ** 2, axis=1, keepdims=True)
    hn = (h - mu) * jax.lax.rsqrt(var + 1e-5) * gamma + beta
    a = hn * (1.0 / (1.0 + jnp.exp(-hn)))
    return stt + _f32dot(a, W2) + b2


def _idx_body(x_ref, Wnp_ref, bnp_ref, idx_ref):
    b = pl.program_id(0)
    n = x_ref.shape[1]
    st = x_ref[0]
    kn = _norm_keys(st, Wnp_ref[...], bnp_ref[...])
    for rt in range(n // _TR):
        knt = kn[rt * _TR:(rt + 1) * _TR]
        idx_ref[0, pl.ds(rt * _TR, _TR), :] = (
            _topk_tile(knt, kn, n, rt * _TR) + b * n)


def _update_idx_body(x_ref, nsum_ref, Wnp_ref, bnp_ref, W1a_ref, W1b_ref,
                     b1_ref, gamma_ref, beta_ref, W2_ref, b2_ref,
                     newstate_ref, idx_ref):
    b = pl.program_id(0)
    n = x_ref.shape[1]
    st = x_ref[0]
    ns = nsum_ref[0]
    tiles = []
    for rt in range(n // _TR):
        sl = slice(rt * _TR, (rt + 1) * _TR)
        tiles.append(_mlp_update(st[sl], ns[sl], Wnp_ref[...], bnp_ref[...],
                                 W1a_ref[...], W1b_ref[...], b1_ref[...],
                                 gamma_ref[...], beta_ref[...], W2_ref[...],
                                 b2_ref[...]))
    newst = jnp.concatenate(tiles, axis=0)
    newstate_ref[0] = newst
    kn = _norm_keys(newst, Wnp_ref[...], bnp_ref[...])
    for rt in range(n // _TR):
        knt = kn[rt * _TR:(rt + 1) * _TR]
        idx_ref[0, pl.ds(rt * _TR, _TR), :] = (
            _topk_tile(knt, kn, n, rt * _TR) + b * n)


def _update_readout_body(x_ref, nsum_ref, Wnp_ref, bnp_ref, W1a_ref, W1b_ref,
                         b1_ref, gamma_ref, beta_ref, W2_ref, b2_ref,
                         Wo_ref, bo_ref, out_ref):
    n = x_ref.shape[1]
    st = x_ref[0]
    ns = nsum_ref[0]
    acc = jnp.zeros((1, st.shape[1]), jnp.float32)
    for rt in range(n // _TR):
        sl = slice(rt * _TR, (rt + 1) * _TR)
        newt = _mlp_update(st[sl], ns[sl], Wnp_ref[...], bnp_ref[...],
                           W1a_ref[...], W1b_ref[...], b1_ref[...],
                           gamma_ref[...], beta_ref[...], W2_ref[...],
                           b2_ref[...])
        acc = acc + jnp.sum(newt, axis=0, keepdims=True)
    out_ref[0] = _f32dot(acc * (1.0 / n), Wo_ref[...]) + bo_ref[...]


def _sc_gather_body(table_hbm, gidx_hbm, out_hbm, gidx_v, rows_v, obuf_v,
                    *sems):
    nch = gidx_v.shape[0]
    k = _TOPK
    per_w = nch * _CHUNK // k
    c = jax.lax.axis_index("c")
    s = jax.lax.axis_index("s")
    wid = s * 2 + c
    base = wid * per_w
    pltpu.sync_copy(gidx_hbm.at[wid], gidx_v)
    copies = [None] * nch

    def fire(ch):
        copies[ch] = pltpu.async_copy(
            table_hbm.at[gidx_v.at[ch]], rows_v.at[ch % _RING],
            sems[ch % _RING])

    for ch in range(min(_RING, nch)):
        fire(ch)
    for ch in range(nch):
        copies[ch].wait()
        rows = rows_v.at[ch % _RING]

        def body(o, carry):
            r0 = o * k
            for col in range(obuf_v.shape[1] // 16):
                sl = pl.ds(col * 16, 16)
                v = rows[r0, sl]
                for j in range(1, k):
                    v = v + rows[r0 + j, sl]
                obuf_v[o, sl] = v
            return carry

        jax.lax.fori_loop(0, _CHUNK // k, body, 0)
        pltpu.sync_copy(obuf_v, out_hbm.at[pl.ds(base + ch * (_CHUNK // k),
                                                 _CHUNK // k)])
        if ch + _RING < nch:
            fire(ch + _RING)


def _sc_gather_sum(table, gidx):
    rows, d = table.shape
    nch = rows * _TOPK // (_NW * _CHUNK)
    mesh = plsc.VectorSubcoreMesh(core_axis_name="c", subcore_axis_name="s")
    return pl.kernel(
        _sc_gather_body,
        mesh=mesh,
        compiler_params=pltpu.CompilerParams(use_tc_tiling_on_sc=False),
        out_type=jax.ShapeDtypeStruct((rows, d), jnp.float32),
        scratch_types=[
            pltpu.VMEM((nch, _CHUNK), jnp.int32),
            pltpu.VMEM((_RING, _CHUNK, d), jnp.float32),
            pltpu.VMEM((_CHUNK // _TOPK, d), jnp.float32),
        ] + [pltpu.SemaphoreType.DMA] * _RING,
    )(table, gidx)


@jax.jit
def kernel(x, W_np, b_np, W1, b1, gamma, beta, W2, b2, Wo, bo):
    B, N, D = x.shape
    H = W_np.shape[1]
    O = Wo.shape[1]
    W1a = W1[:D]
    W1b = W1[D:]
    row = lambda v: v.reshape(1, -1)
    bnp, b1r, gr, br, b2r, bor = (row(b_np), row(b1), row(gamma), row(beta),
                                  row(b2), row(bo))

    full = lambda shape: pl.BlockSpec(shape, lambda b: (0,) * len(shape))
    bspec = lambda shape: pl.BlockSpec((1,) + shape,
                                       lambda b: (b,) + (0,) * len(shape))
    wspecs = [full((D, H)), full((1, H)), full((D, H)), full((H, H)),
              full((1, H)), full((1, H)), full((1, H)), full((H, D)),
              full((1, D))]

    idx_call = pl.pallas_call(
        _idx_body, grid=(B,),
        in_specs=[bspec((N, D)), full((D, H)), full((1, H))],
        out_specs=bspec((N, _TOPK)),
        out_shape=jax.ShapeDtypeStruct((B, N, _TOPK), jnp.int32),
    )
    upd_idx_call = pl.pallas_call(
        _update_idx_body, grid=(B,),
        in_specs=[bspec((N, D)), bspec((N, D))] + wspecs,
        out_specs=[bspec((N, D)), bspec((N, _TOPK))],
        out_shape=[jax.ShapeDtypeStruct((B, N, D), jnp.float32),
                   jax.ShapeDtypeStruct((B, N, _TOPK), jnp.int32)],
    )
    upd_out_call = pl.pallas_call(
        _update_readout_body, grid=(B,),
        in_specs=[bspec((N, D)), bspec((N, D))] + wspecs +
                 [full((D, O)), full((1, O))],
        out_specs=bspec((1, O)),
        out_shape=jax.ShapeDtypeStruct((B, 1, O), jnp.float32),
    )

    def gather(state, idx):
        gidx = idx.reshape(_NW, B * N * _TOPK // (_NW * _CHUNK), _CHUNK)
        nsum = _sc_gather_sum(state.reshape(B * N, D), gidx)
        return nsum.reshape(B, N, D)

    state = x
    idx = idx_call(x, W_np, bnp)
    for _step in range(_NUM_STEPS - 1):
        nsum = gather(state, idx)
        state, idx = upd_idx_call(state, nsum, W_np, bnp, W1a, W1b, b1r,
                                  gr, br, W2, b2r)
    nsum = gather(state, idx)
    out = upd_out_call(state, nsum, W_np, bnp, W1a, W1b, b1r, gr, br, W2,
                       b2r, Wo, bor)
    return out.reshape(B, O)


# speedup vs baseline: 3.8146x; 1.9400x over previous
import functools

import jax
import jax.numpy as jnp
from jax.experimental import pallas as pl
from jax.experimental.pallas import tpu as pltpu
from jax.experimental.pallas import tpu_sc as plsc

_NUM_STEPS = 3
_TOPK = 8
_TR = 512

_NW = 32
_CHUNK = 128
_RING = 8


def _f32dot(a, b):
    return jax.lax.dot_general(a, b, (((1,), (0,)), ((), ())),
                               preferred_element_type=jnp.float32)


def _norm_keys(st, Wnp, bnp):
    keys = _f32dot(st, Wnp) + bnp
    nrm = jnp.sqrt(jnp.sum(keys * keys, axis=1, keepdims=True))
    return keys / jnp.maximum(nrm, 1e-12)


def _topk_tile(knt, kn, n, base):
    sim = jax.lax.dot_general(knt, kn, (((1,), (1,)), ((), ())),
                              preferred_element_type=jnp.float32)
    iota = jax.lax.broadcasted_iota(jnp.int32, sim.shape, 1)
    iota_f = iota.astype(jnp.float32)
    self_col = jax.lax.broadcasted_iota(jnp.int32, (sim.shape[0], 1), 0) + base
    s = jnp.where(iota == self_col, -jnp.inf, sim)
    fn = jnp.float32(n)
    cols = [self_col]
    for _j in range(_TOPK - 1):
        m = jnp.max(s, axis=1, keepdims=True)
        hit = s == m
        am_f = jnp.min(jnp.where(hit, iota_f, fn), axis=1, keepdims=True)
        cols.append(am_f.astype(jnp.int32))
        if _j + 2 < _TOPK:
            s = jnp.where(hit, -jnp.inf, s)
    return jnp.concatenate(cols, axis=1)


def _mlp_update(stt, nsum_t, Wnp, bnp, W1a, W1b, b1, gamma, beta, W2, b2):
    agg = _f32dot(nsum_t * (1.0 / _TOPK), Wnp) + bnp
    h = _f32dot(stt, W1a) + _f32dot(agg, W1b) + b1
    mu = jnp.mean(h, axis=1, keepdims=True)
    var = jnp.mean((h - mu) ** 2, axis=1, keepdims=True)
    hn = (h - mu) * jax.lax.rsqrt(var + 1e-5) * gamma + beta
    a = hn * (1.0 / (1.0 + jnp.exp(-hn)))
    return stt + _f32dot(a, W2) + b2


def _idx_body(x_ref, Wnp_ref, bnp_ref, idx_ref):
    b = pl.program_id(0)
    n = x_ref.shape[1]
    st = x_ref[0]
    kn = _norm_keys(st, Wnp_ref[...], bnp_ref[...])
    for rt in range(n // _TR):
        knt = kn[rt * _TR:(rt + 1) * _TR]
        idx_ref[0, pl.ds(rt * _TR, _TR), :] = (
            _topk_tile(knt, kn, n, rt * _TR) + b * n)


def _update_idx_body(x_ref, nsum_ref, Wnp_ref, bnp_ref, W1a_ref, W1b_ref,
                     b1_ref, gamma_ref, beta_ref, W2_ref, b2_ref,
                     newstate_ref, idx_ref):
    b = pl.program_id(0)
    n = x_ref.shape[1]
    st = x_ref[0]
    ns = nsum_ref[0]
    tiles = []
    for rt in range(n // _TR):
        sl = slice(rt * _TR, (rt + 1) * _TR)
        tiles.append(_mlp_update(st[sl], ns[sl], Wnp_ref[...], bnp_ref[...],
                                 W1a_ref[...], W1b_ref[...], b1_ref[...],
                                 gamma_ref[...], beta_ref[...], W2_ref[...],
                                 b2_ref[...]))
    newst = jnp.concatenate(tiles, axis=0)
    newstate_ref[0] = newst
    kn = _norm_keys(newst, Wnp_ref[...], bnp_ref[...])
    for rt in range(n // _TR):
        knt = kn[rt * _TR:(rt + 1) * _TR]
        idx_ref[0, pl.ds(rt * _TR, _TR), :] = (
            _topk_tile(knt, kn, n, rt * _TR) + b * n)


def _update_readout_body(x_ref, nsum_ref, Wnp_ref, bnp_ref, W1a_ref, W1b_ref,
                         b1_ref, gamma_ref, beta_ref, W2_ref, b2_ref,
                         Wo_ref, bo_ref, out_ref):
    n = x_ref.shape[1]
    st = x_ref[0]
    ns = nsum_ref[0]
    acc = jnp.zeros((1, st.shape[1]), jnp.float32)
    for rt in range(n // _TR):
        sl = slice(rt * _TR, (rt + 1) * _TR)
        newt = _mlp_update(st[sl], ns[sl], Wnp_ref[...], bnp_ref[...],
                           W1a_ref[...], W1b_ref[...], b1_ref[...],
                           gamma_ref[...], beta_ref[...], W2_ref[...],
                           b2_ref[...])
        acc = acc + jnp.sum(newt, axis=0, keepdims=True)
    out_ref[0] = _f32dot(acc * (1.0 / n), Wo_ref[...]) + bo_ref[...]


def _sc_gather_body(table_hbm, gidx_hbm, out_hbm, gidx_v, rows_v, obuf_v,
                    *sems):
    nch = gidx_v.shape[0]
    k = _TOPK
    per_w = nch * _CHUNK // k
    c = jax.lax.axis_index("c")
    s = jax.lax.axis_index("s")
    wid = s * 2 + c
    base = wid * per_w
    pltpu.sync_copy(gidx_hbm.at[wid], gidx_v)
    copies = [None] * nch

    def fire(ch):
        copies[ch] = pltpu.async_copy(
            table_hbm.at[gidx_v.at[ch]], rows_v.at[ch % _RING],
            sems[ch % _RING])

    for ch in range(min(_RING, nch)):
        fire(ch)
    for ch in range(nch):
        copies[ch].wait()
        rows = rows_v.at[ch % _RING]

        def body(o, carry):
            r0 = o * k
            for col in range(obuf_v.shape[1] // 16):
                sl = pl.ds(col * 16, 16)
                v = rows[r0, sl]
                for j in range(1, k):
                    v = v + rows[r0 + j, sl]
                obuf_v[o, sl] = v
            return carry

        jax.lax.fori_loop(0, _CHUNK // k, body, 0)
        pltpu.sync_copy(obuf_v, out_hbm.at[pl.ds(base + ch * (_CHUNK // k),
                                                 _CHUNK // k)])
        if ch + _RING < nch:
            fire(ch + _RING)


def _sc_gather_sum(table, gidx):
    rows, d = table.shape
    nch = rows * _TOPK // (_NW * _CHUNK)
    mesh = plsc.VectorSubcoreMesh(core_axis_name="c", subcore_axis_name="s")
    return pl.kernel(
        _sc_gather_body,
        mesh=mesh,
        compiler_params=pltpu.CompilerParams(use_tc_tiling_on_sc=False),
        out_type=jax.ShapeDtypeStruct((rows, d), jnp.float32),
        scratch_types=[
            pltpu.VMEM((nch, _CHUNK), jnp.int32),
            pltpu.VMEM((_RING, _CHUNK, d), jnp.float32),
            pltpu.VMEM((_CHUNK // _TOPK, d), jnp.float32),
        ] + [pltpu.SemaphoreType.DMA] * _RING,
    )(table, gidx)


@jax.jit
def kernel(x, W_np, b_np, W1, b1, gamma, beta, W2, b2, Wo, bo):
    B, N, D = x.shape
    H = W_np.shape[1]
    O = Wo.shape[1]
    W1a = W1[:D]
    W1b = W1[D:]
    row = lambda v: v.reshape(1, -1)
    bnp, b1r, gr, br, b2r, bor = (row(b_np), row(b1), row(gamma), row(beta),
                                  row(b2), row(bo))

    full = lambda shape: pl.BlockSpec(shape, lambda b: (0,) * len(shape))
    bspec = lambda shape: pl.BlockSpec((1,) + shape,
                                       lambda b: (b,) + (0,) * len(shape))
    wspecs = [full((D, H)), full((1, H)), full((D, H)), full((H, H)),
              full((1, H)), full((1, H)), full((1, H)), full((H, D)),
              full((1, D))]

    idx_call = pl.pallas_call(
        _idx_body, grid=(B,),
        in_specs=[bspec((N, D)), full((D, H)), full((1, H))],
        out_specs=bspec((N, _TOPK)),
        out_shape=jax.ShapeDtypeStruct((B, N, _TOPK), jnp.int32),
    )
    upd_idx_call = pl.pallas_call(
        _update_idx_body, grid=(B,),
        in_specs=[bspec((N, D)), bspec((N, D))] + wspecs,
        out_specs=[bspec((N, D)), bspec((N, _TOPK))],
        out_shape=[jax.ShapeDtypeStruct((B, N, D), jnp.float32),
                   jax.ShapeDtypeStruct((B, N, _TOPK), jnp.int32)],
    )
    upd_out_call = pl.pallas_call(
        _update_readout_body, grid=(B,),
        in_specs=[bspec((N, D)), bspec((N, D))] + wspecs +
                 [full((D, O)), full((1, O))],
        out_specs=bspec((1, O)),
        out_shape=jax.ShapeDtypeStruct((B, 1, O), jnp.float32),
    )

    def gather(state, idx):
        gidx = idx.reshape(_NW, B * N * _TOPK // (_NW * _CHUNK), _CHUNK)
        nsum = _sc_gather_sum(state.reshape(B * N, D), gidx)
        return nsum.reshape(B, N, D)

    state = x
    idx = idx_call(x, W_np, bnp)
    for _step in range(_NUM_STEPS - 1):
        nsum = gather(state, idx)
        state, idx = upd_idx_call(state, nsum, W_np, bnp, W1a, W1b, b1r,
                                  gr, br, W2, b2r)
    nsum = gather(state, idx)
    out = upd_out_call(state, nsum, W_np, bnp, W1a, W1b, b1r, gr, br, W2,
                       b2r, Wo, bor)
    return out.reshape(B, O)
